# Initial kernel scaffold; baseline (speedup 1.0000x reference)
import jax
import jax.numpy as jnp
from jax.experimental import pallas as pl


def kernel(x, pos, batch, params):
    raise NotImplementedError("write your pallas kernel here")


# speedup vs baseline: 1.0390x; 1.0390x over previous
import functools
import jax
import jax.numpy as jnp
from jax.experimental import pallas as pl
from jax.experimental.pallas import tpu as pltpu

CHUNK = 128


def _lrelu(v, s):
    return jnp.where(v >= 0, v, s * v)


def _head_kernel(h_ref, w1_ref, b1_ref, w2_ref, b2_ref, w3_ref, b3_ref, o_ref):
    h = h_ref[...]
    a = jnp.maximum(h @ w1_ref[...].T + b1_ref[...], 0.0)
    b = a @ w2_ref[...].T + b2_ref[...]
    o_ref[...] = b @ w3_ref[...].T + b3_ref[...]


def _head(h, params):
    n = h.shape[0]
    blk = 4096
    w1, b1 = params["h1_W"], params["h1_b"]
    w2, b2 = params["h2_W"], params["h2_b"]
    w3, b3 = params["lin_W"], params["lin_b"]
    out = pl.pallas_call(
        _head_kernel,
        grid=(n // blk,),
        in_specs=[
            pl.BlockSpec((blk, h.shape[1]), lambda i: (i, 0)),
            pl.BlockSpec(w1.shape, lambda i: (0, 0)),
            pl.BlockSpec(b1.shape, lambda i: (0,)),
            pl.BlockSpec(w2.shape, lambda i: (0, 0)),
            pl.BlockSpec(b2.shape, lambda i: (0,)),
            pl.BlockSpec(w3.shape, lambda i: (0, 0)),
            pl.BlockSpec(b3.shape, lambda i: (0,)),
        ],
        out_specs=pl.BlockSpec((blk, w3.shape[0]), lambda i: (i, 0)),
        out_shape=jax.ShapeDtypeStruct((n, w3.shape[0]), h.dtype),
    )(h, w1, b1, w2, b2, w3, b3)
    return out


def _knn_idx(q, s, k):
    def f(qc):
        d = jnp.sum((qc[:, None, :] - s[None, :, :]) ** 2, axis=-1)
        return jax.lax.top_k(-d, k)[1]
    return jax.lax.map(f, q.reshape(-1, CHUNK, q.shape[1])).reshape(q.shape[0], k)


def _nn1(q, s):
    def f(qc):
        d = jnp.sum((qc[:, None, :] - s[None, :, :]) ** 2, axis=-1)
        return jnp.argmin(d, axis=1)
    return jax.lax.map(f, q.reshape(-1, CHUNK, q.shape[1])).reshape(q.shape[0])


def _lfa(params, pfx, col, xf, pos_all, n_q, k):
    x_j = xf[col]
    pos_i = jnp.repeat(pos_all[:n_q], k, axis=0)
    pos_j = pos_all[col]
    dist = pos_j - pos_i
    eu = jnp.sum(jnp.abs(dist), axis=1, keepdims=True)
    rel = jnp.concatenate([pos_i, pos_j, dist, eu], axis=1)
    lse = rel @ params[pfx + "e_W"].T + params[pfx + "e_b"]
    out1 = jnp.concatenate([x_j, lse], axis=1)
    att = jax.nn.softmax(out1 @ params[pfx + "a_W"].T + params[pfx + "a_b"], axis=-1)
    msg = att * out1
    return msg.reshape(n_q, k, msg.shape[1]).sum(axis=1)


def _block(params, pfx, x, pos, decimation, k):
    n = x.shape[0]
    idx = jnp.arange(0, n, decimation)
    n_q = idx.shape[0]
    q_pos = pos[idx]
    nbrs = _knn_idx(q_pos, pos, k)
    col = nbrs.reshape(-1)

    h0 = _lrelu(x @ params[pfx + "_m1_W"].T + params[pfx + "_m1_b"], 0.2)
    h1 = _lfa(params, pfx + "_l1_", col, h0, pos, n_q, k)
    h1_full_gather = jnp.where((col < n_q)[:, None], h1[jnp.minimum(col, n_q - 1)], 0.0)
    h2 = _lfa_pregathered(params, pfx + "_l2_", col, h1_full_gather, pos, n_q, k)
    h2_idx = jnp.where((idx < n_q)[:, None], h2[jnp.minimum(idx, n_q - 1)], 0.0)
    m2 = _lrelu(h2_idx @ params[pfx + "_m2_W"].T + params[pfx + "_m2_b"], 0.2)
    sc = _lrelu(x[idx] @ params[pfx + "_sc_W"].T + params[pfx + "_sc_b"], 0.2)
    out = _lrelu(m2 + sc, 0.01)
    return out, q_pos


def _lfa_pregathered(params, pfx, col, x_j, pos_all, n_q, k):
    pos_i = jnp.repeat(pos_all[:n_q], k, axis=0)
    pos_j = pos_all[col]
    dist = pos_j - pos_i
    eu = jnp.sum(jnp.abs(dist), axis=1, keepdims=True)
    rel = jnp.concatenate([pos_i, pos_j, dist, eu], axis=1)
    lse = rel @ params[pfx + "e_W"].T + params[pfx + "e_b"]
    out1 = jnp.concatenate([x_j, lse], axis=1)
    att = jax.nn.softmax(out1 @ params[pfx + "a_W"].T + params[pfx + "a_b"], axis=-1)
    msg = att * out1
    return msg.reshape(n_q, k, msg.shape[1]).sum(axis=1)


def _fp(params, pfx, xh, pos, pos_skip, x_skip):
    nn = _nn1(pos_skip, pos)
    xi = xh[nn]
    if x_skip is not None:
        xi = jnp.concatenate([xi, x_skip], axis=1)
    return xi @ params[pfx + "_W"].T + params[pfx + "_b"]


@jax.jit
def _forward(x, pos, params):
    x0, p0 = x, pos
    x1, p1 = _block(params, "b1", x0, p0, 4, 16)
    x2, p2 = _block(params, "b2", x1, p1, 4, 16)
    x3, p3 = _block(params, "b3", x2, p2, 4, 16)
    x4, p4 = _block(params, "b4", x3, p3, 4, 16)
    h = x4 @ params["mlp1_W"].T + params["mlp1_b"]
    h = _fp(params, "fp4", h, p4, p3, x3)
    h = _fp(params, "fp3", h, p3, p2, x2)
    h = _fp(params, "fp2", h, p2, p1, x1)
    h = _fp(params, "fp1", h, p1, p0, x0)
    return _head(h, params)


def kernel(x, pos, batch, params):
    return _forward(x, pos, params)


# speedup vs baseline: 2.7681x; 2.6643x over previous
import functools
import jax
import jax.numpy as jnp
from jax.experimental import pallas as pl
from jax.experimental.pallas import tpu as pltpu

CHUNK = 128


def _lrelu(v, s):
    return jnp.where(v >= 0, v, s * v)


def _head_kernel(h_ref, w1_ref, b1_ref, w2_ref, b2_ref, w3_ref, b3_ref, o_ref):
    h = h_ref[...]
    a = jnp.maximum(h @ w1_ref[...].T + b1_ref[...], 0.0)
    b = a @ w2_ref[...].T + b2_ref[...]
    o_ref[...] = b @ w3_ref[...].T + b3_ref[...]


def _head(h, params):
    n = h.shape[0]
    blk = 4096
    w1, b1 = params["h1_W"], params["h1_b"]
    w2, b2 = params["h2_W"], params["h2_b"]
    w3, b3 = params["lin_W"], params["lin_b"]
    out = pl.pallas_call(
        _head_kernel,
        grid=(n // blk,),
        in_specs=[
            pl.BlockSpec((blk, h.shape[1]), lambda i: (i, 0)),
            pl.BlockSpec(w1.shape, lambda i: (0, 0)),
            pl.BlockSpec(b1.shape, lambda i: (0,)),
            pl.BlockSpec(w2.shape, lambda i: (0, 0)),
            pl.BlockSpec(b2.shape, lambda i: (0,)),
            pl.BlockSpec(w3.shape, lambda i: (0, 0)),
            pl.BlockSpec(b3.shape, lambda i: (0,)),
        ],
        out_specs=pl.BlockSpec((blk, w3.shape[0]), lambda i: (i, 0)),
        out_shape=jax.ShapeDtypeStruct((n, w3.shape[0]), h.dtype),
    )(h, w1, b1, w2, b2, w3, b3)
    return out


_I32_MAX = jnp.iinfo(jnp.int32).max


def _aug_q(p):
    n2 = jnp.sum(p * p, axis=1, keepdims=True)
    one = jnp.ones_like(n2)
    zero = jnp.zeros((p.shape[0], 3), p.dtype)
    return jnp.concatenate([p, n2, one, zero], axis=1)


def _aug_s(p):
    n2 = jnp.sum(p * p, axis=1, keepdims=True)
    one = jnp.ones_like(n2)
    zero = jnp.zeros((p.shape[0], 3), p.dtype)
    return jnp.concatenate([-2.0 * p, one, n2, zero], axis=1)


def _knn16_body(q_ref, s_ref, o_ref, d_ref):
    tq = q_ref.shape[0]
    s_cols = s_ref.shape[0]
    nv = s_cols // 128
    d_ref[...] = jax.lax.dot_general(q_ref[...], s_ref[...], (((1,), (1,)), ((), ())),
                                     preferred_element_type=jnp.float32,
                                     precision=jax.lax.Precision.HIGHEST)

    init = tuple(jnp.full((tq, 128), _I32_MAX, jnp.int32) for _ in range(4))

    def body(j, ms):
        m1, m2, m3, m4 = ms
        x = jnp.maximum(d_ref[:, pl.ds(j * 128, 128)], 0.0)
        x = jax.lax.bitcast_convert_type(x, jnp.int32)
        x = (x & ~0xFF) | j
        t = jnp.minimum(m1, x); x = jnp.maximum(m1, x); m1 = t
        t = jnp.minimum(m2, x); x = jnp.maximum(m2, x); m2 = t
        t = jnp.minimum(m3, x); x = jnp.maximum(m3, x); m3 = t
        m4 = jnp.minimum(m4, x)
        return (m1, m2, m3, m4)

    ms = jax.lax.fori_loop(0, nv, body, init)
    cand = jnp.concatenate(ms, axis=1)
    iota = jax.lax.broadcasted_iota(jnp.int32, (tq, 512), 1)
    for kk in range(16):
        mn = jnp.min(cand, axis=1, keepdims=True)
        pos = jnp.min(jnp.where(cand == mn, iota, _I32_MAX), axis=1, keepdims=True)
        col = ((mn & 0xFF) << 7) | (pos & 127)
        o_ref[:, kk:kk + 1] = col
        cand = jnp.where(iota == pos, _I32_MAX, cand)


def _knn_idx(q, s, k):
    assert k == 16
    qn, sn = q.shape[0], s.shape[0]
    tq = min(qn, max(64, (8 * 1024 * 1024 // (4 * sn)) // 64 * 64))
    qa, sa = _aug_q(q), _aug_s(s)
    from jax.experimental.pallas import tpu as pltpu_mod
    out = pl.pallas_call(
        _knn16_body,
        grid=(qn // tq,),
        in_specs=[
            pl.BlockSpec((tq, 8), lambda i: (i, 0)),
            pl.BlockSpec((sn, 8), lambda i: (0, 0)),
        ],
        out_specs=pl.BlockSpec((tq, 16), lambda i: (i, 0)),
        out_shape=jax.ShapeDtypeStruct((qn, 16), jnp.int32),
        scratch_shapes=[pltpu_mod.VMEM((tq, sn), jnp.float32)],
    )(qa, sa)
    return out


def _nn1_body(q_ref, s_ref, o_ref, d_ref):
    tq = q_ref.shape[0]
    s_cols = s_ref.shape[0]
    nv = s_cols // 128
    d_ref[...] = jax.lax.dot_general(q_ref[...], s_ref[...], (((1,), (1,)), ((), ())),
                                     preferred_element_type=jnp.float32,
                                     precision=jax.lax.Precision.HIGHEST)

    mv0 = jnp.full((tq, 128), jnp.inf, jnp.float32)
    mi0 = jnp.zeros((tq, 128), jnp.int32)

    def body(j, ms):
        mv, mi = ms
        x = d_ref[:, pl.ds(j * 128, 128)]
        upd = x < mv
        return jnp.where(upd, x, mv), jnp.where(upd, j, mi)

    mv, mi = jax.lax.fori_loop(0, nv, body, (mv0, mi0))
    v = jnp.min(mv, axis=1, keepdims=True)
    lane = jax.lax.broadcasted_iota(jnp.int32, (tq, 128), 1)
    colf = (mi << 7) | lane
    col = jnp.min(jnp.where(mv == v, colf, _I32_MAX), axis=1, keepdims=True)
    o_ref[...] = col


def _nn1(q, s):
    qn, sn = q.shape[0], s.shape[0]
    tq = min(qn, max(64, (8 * 1024 * 1024 // (4 * sn)) // 64 * 64))
    if qn % tq:
        tq = 64 if qn % 64 == 0 else qn
    qa, sa = _aug_q(q), _aug_s(s)
    from jax.experimental.pallas import tpu as pltpu_mod
    out = pl.pallas_call(
        _nn1_body,
        grid=(qn // tq,),
        in_specs=[
            pl.BlockSpec((tq, 8), lambda i: (i, 0)),
            pl.BlockSpec((sn, 8), lambda i: (0, 0)),
        ],
        out_specs=pl.BlockSpec((tq, 1), lambda i: (i, 0)),
        out_shape=jax.ShapeDtypeStruct((qn, 1), jnp.int32),
        scratch_shapes=[pltpu_mod.VMEM((tq, sn), jnp.float32)],
    )(qa, sa)
    return out.reshape(qn)


def _lfa(params, pfx, col, xf, pos_all, n_q, k):
    x_j = xf[col]
    pos_i = jnp.repeat(pos_all[:n_q], k, axis=0)
    pos_j = pos_all[col]
    dist = pos_j - pos_i
    eu = jnp.sum(jnp.abs(dist), axis=1, keepdims=True)
    rel = jnp.concatenate([pos_i, pos_j, dist, eu], axis=1)
    lse = rel @ params[pfx + "e_W"].T + params[pfx + "e_b"]
    out1 = jnp.concatenate([x_j, lse], axis=1)
    att = jax.nn.softmax(out1 @ params[pfx + "a_W"].T + params[pfx + "a_b"], axis=-1)
    msg = att * out1
    return msg.reshape(n_q, k, msg.shape[1]).sum(axis=1)


def _block(params, pfx, x, pos, decimation, k):
    n = x.shape[0]
    idx = jnp.arange(0, n, decimation)
    n_q = idx.shape[0]
    q_pos = pos[idx]
    nbrs = _knn_idx(q_pos, pos, k)
    col = nbrs.reshape(-1)

    h0 = _lrelu(x @ params[pfx + "_m1_W"].T + params[pfx + "_m1_b"], 0.2)
    h1 = _lfa(params, pfx + "_l1_", col, h0, pos, n_q, k)
    h1_full_gather = jnp.where((col < n_q)[:, None], h1[jnp.minimum(col, n_q - 1)], 0.0)
    h2 = _lfa_pregathered(params, pfx + "_l2_", col, h1_full_gather, pos, n_q, k)
    h2_idx = jnp.where((idx < n_q)[:, None], h2[jnp.minimum(idx, n_q - 1)], 0.0)
    m2 = _lrelu(h2_idx @ params[pfx + "_m2_W"].T + params[pfx + "_m2_b"], 0.2)
    sc = _lrelu(x[idx] @ params[pfx + "_sc_W"].T + params[pfx + "_sc_b"], 0.2)
    out = _lrelu(m2 + sc, 0.01)
    return out, q_pos


def _lfa_pregathered(params, pfx, col, x_j, pos_all, n_q, k):
    pos_i = jnp.repeat(pos_all[:n_q], k, axis=0)
    pos_j = pos_all[col]
    dist = pos_j - pos_i
    eu = jnp.sum(jnp.abs(dist), axis=1, keepdims=True)
    rel = jnp.concatenate([pos_i, pos_j, dist, eu], axis=1)
    lse = rel @ params[pfx + "e_W"].T + params[pfx + "e_b"]
    out1 = jnp.concatenate([x_j, lse], axis=1)
    att = jax.nn.softmax(out1 @ params[pfx + "a_W"].T + params[pfx + "a_b"], axis=-1)
    msg = att * out1
    return msg.reshape(n_q, k, msg.shape[1]).sum(axis=1)


def _fp(params, pfx, xh, pos, pos_skip, x_skip):
    nn = _nn1(pos_skip, pos)
    xi = xh[nn]
    if x_skip is not None:
        xi = jnp.concatenate([xi, x_skip], axis=1)
    return xi @ params[pfx + "_W"].T + params[pfx + "_b"]


@jax.jit
def _forward(x, pos, params):
    x0, p0 = x, pos
    x1, p1 = _block(params, "b1", x0, p0, 4, 16)
    x2, p2 = _block(params, "b2", x1, p1, 4, 16)
    x3, p3 = _block(params, "b3", x2, p2, 4, 16)
    x4, p4 = _block(params, "b4", x3, p3, 4, 16)
    h = x4 @ params["mlp1_W"].T + params["mlp1_b"]
    h = _fp(params, "fp4", h, p4, p3, x3)
    h = _fp(params, "fp3", h, p3, p2, x2)
    h = _fp(params, "fp2", h, p2, p1, x1)
    h = _fp(params, "fp1", h, p1, p0, x0)
    return _head(h, params)


def kernel(x, pos, batch, params):
    return _forward(x, pos, params)
